# Initial kernel scaffold; baseline (speedup 1.0000x reference)
import jax
import jax.numpy as jnp
from jax.experimental import pallas as pl


def kernel(indices, table):
    raise NotImplementedError("write your pallas kernel here")


# speedup vs baseline: 2.9723x; 2.9723x over previous
import functools

import jax
import jax.numpy as jnp
from jax import lax
from jax.experimental import pallas as pl
from jax.experimental.pallas import tpu as pltpu, tpu_sc as plsc

VOCAB = 100000
DIM = 128

N_CORES = 2
N_SUBCORES = 16
N_WORKERS = N_CORES * N_SUBCORES

TOTAL = 4096 * 50
ROWS_PER_W = TOTAL // N_WORKERS
CHUNK = 128
N_CHUNKS = ROWS_PER_W // CHUNK

_MESH = plsc.VectorSubcoreMesh(core_axis_name="c", subcore_axis_name="s")


@functools.partial(
    pl.kernel,
    out_type=jax.ShapeDtypeStruct((TOTAL, DIM), jnp.float32),
    mesh=_MESH,
    scratch_types=[
        pltpu.VMEM((N_CHUNKS, CHUNK), jnp.int32),
        pltpu.VMEM((CHUNK, DIM), jnp.float32),
        pltpu.SemaphoreType.DMA,
    ],
)
def _gather_kernel(idx_hbm, table_hbm, out_hbm, idx_v, rows_v, sem):
    wid = lax.axis_index("s") * N_CORES + lax.axis_index("c")
    base = wid * ROWS_PER_W
    pltpu.sync_copy(idx_hbm.at[wid], idx_v)

    def chunk_body(c, _):
        pltpu.async_copy(table_hbm.at[idx_v.at[c]], rows_v, sem).wait()
        pltpu.sync_copy(rows_v, out_hbm.at[pl.ds(base + c * CHUNK, CHUNK)])
        return 0

    lax.fori_loop(0, N_CHUNKS, chunk_body, 0, unroll=False)


def kernel(indices, table):
    idx = indices.reshape(N_WORKERS, N_CHUNKS, CHUNK).astype(jnp.int32)
    out = _gather_kernel(idx, table)
    return out.reshape(indices.shape + (DIM,))


# speedup vs baseline: 3.3462x; 1.1258x over previous
import functools

import jax
import jax.numpy as jnp
from jax import lax
from jax.experimental import pallas as pl
from jax.experimental.pallas import tpu as pltpu, tpu_sc as plsc

VOCAB = 100000
DIM = 128

N_CORES = 2
N_SUBCORES = 16
N_WORKERS = N_CORES * N_SUBCORES

TOTAL = 4096 * 50
ROWS_PER_W = TOTAL // N_WORKERS
CHUNK = 128
N_CHUNKS = ROWS_PER_W // CHUNK
NBUF = 4
SKEW = 2

_MESH = plsc.VectorSubcoreMesh(core_axis_name="c", subcore_axis_name="s")


@functools.partial(
    pl.kernel,
    out_type=jax.ShapeDtypeStruct((TOTAL, DIM), jnp.float32),
    mesh=_MESH,
    scratch_types=[
        pltpu.VMEM((N_CHUNKS, CHUNK), jnp.int32),
        pltpu.VMEM((NBUF, CHUNK, DIM), jnp.float32),
        pltpu.SemaphoreType.DMA((NBUF,)),
        pltpu.SemaphoreType.DMA((NBUF,)),
    ],
)
def _gather_kernel(idx_hbm, table_hbm, out_hbm, idx_v, rows_v, gsem, ssem):
    wid = lax.axis_index("s") * N_CORES + lax.axis_index("c")
    base = wid * ROWS_PER_W
    pltpu.sync_copy(idx_hbm.at[wid], idx_v)

    def gather(c):
        b = lax.rem(c, NBUF) if not isinstance(c, int) else c % NBUF
        return pltpu.make_async_copy(
            table_hbm.at[idx_v.at[c]], rows_v.at[b], gsem.at[b])

    def store(c):
        b = lax.rem(c, NBUF) if not isinstance(c, int) else c % NBUF
        return pltpu.make_async_copy(
            rows_v.at[b], out_hbm.at[pl.ds(base + c * CHUNK, CHUNK)],
            ssem.at[b])

    for i in range(SKEW):
        gather(i).start()
    for i in range(SKEW, NBUF):
        gather(i - SKEW).wait()
        store(i - SKEW).start()
        gather(i).start()

    def steady(i, _):
        store(i - NBUF).wait()
        gather(i - SKEW).wait()
        store(i - SKEW).start()
        gather(i).start()
        return 0

    lax.fori_loop(NBUF, N_CHUNKS, steady, 0, unroll=False)

    for i in range(N_CHUNKS, N_CHUNKS + SKEW):
        store(i - NBUF).wait()
        gather(i - SKEW).wait()
        store(i - SKEW).start()
    for c in range(N_CHUNKS - NBUF + SKEW, N_CHUNKS):
        store(c).wait()


def kernel(indices, table):
    idx = indices.reshape(N_WORKERS, N_CHUNKS, CHUNK).astype(jnp.int32)
    out = _gather_kernel(idx, table)
    return out.reshape(indices.shape + (DIM,))


# speedup vs baseline: 5.0701x; 1.5152x over previous
import functools

import jax
import jax.numpy as jnp
from jax import lax
from jax.experimental import pallas as pl
from jax.experimental.pallas import tpu as pltpu, tpu_sc as plsc

VOCAB = 100000
DIM = 128

N_CORES = 2
N_SUBCORES = 16
N_WORKERS = N_CORES * N_SUBCORES

N_SENT = 4096
SENT_LEN = 50
SENT_PAD = 56
SENT_PER_W = N_SENT // N_WORKERS
PAIRS_PER_W = SENT_PER_W // 2
NBUF = 4
SKEW = 2

_MESH = plsc.VectorSubcoreMesh(core_axis_name="c", subcore_axis_name="s")


@functools.partial(
    pl.kernel,
    out_type=jax.ShapeDtypeStruct((N_SENT * SENT_PAD, DIM), jnp.float32),
    mesh=_MESH,
    scratch_types=[
        pltpu.VMEM((SENT_PER_W, SENT_LEN), jnp.int32),
        pltpu.VMEM((NBUF, 2 * SENT_PAD, DIM), jnp.float32),
        pltpu.SemaphoreType.DMA((NBUF,)),
        pltpu.SemaphoreType.DMA((NBUF,)),
    ],
)
def _gather_kernel(idx_hbm, table_hbm, out_hbm, idx_v, rows_v, gsem, ssem):
    wid = lax.axis_index("s") * N_CORES + lax.axis_index("c")
    sent_base = wid * SENT_PER_W
    pltpu.sync_copy(idx_hbm.at[pl.ds(sent_base, SENT_PER_W)], idx_v)

    def gathers(p):
        b = lax.rem(p, NBUF) if not isinstance(p, int) else p % NBUF
        return [
            pltpu.make_async_copy(
                table_hbm.at[idx_v.at[2 * p + k]],
                rows_v.at[b, pl.ds(k * SENT_PAD, SENT_LEN)],
                gsem.at[b])
            for k in range(2)
        ]

    def store(p):
        b = lax.rem(p, NBUF) if not isinstance(p, int) else p % NBUF
        return pltpu.make_async_copy(
            rows_v.at[b],
            out_hbm.at[pl.ds((sent_base + 2 * p) * SENT_PAD, 2 * SENT_PAD)],
            ssem.at[b])

    def start_gathers(p):
        for g in gathers(p):
            g.start()

    def wait_gathers(p):
        for g in gathers(p):
            g.wait()

    for i in range(SKEW):
        start_gathers(i)
    for i in range(SKEW, NBUF):
        wait_gathers(i - SKEW)
        store(i - SKEW).start()
        start_gathers(i)

    def steady(i, _):
        store(i - NBUF).wait()
        wait_gathers(i - SKEW)
        store(i - SKEW).start()
        start_gathers(i)
        return 0

    lax.fori_loop(NBUF, PAIRS_PER_W, steady, 0, unroll=False)

    for i in range(PAIRS_PER_W, PAIRS_PER_W + SKEW):
        store(i - NBUF).wait()
        wait_gathers(i - SKEW)
        store(i - SKEW).start()
    for p in range(PAIRS_PER_W - NBUF + SKEW, PAIRS_PER_W):
        store(p).wait()


def kernel(indices, table):
    idx = indices.astype(jnp.int32)
    out = _gather_kernel(idx, table)
    return out.reshape(N_SENT, SENT_PAD, DIM)[:, :SENT_LEN, :]


# speedup vs baseline: 5.0897x; 1.0039x over previous
import functools

import jax
import jax.numpy as jnp
from jax import lax
from jax.experimental import pallas as pl
from jax.experimental.pallas import tpu as pltpu, tpu_sc as plsc

VOCAB = 100000
DIM = 128

N_CORES = 2
N_SUBCORES = 16
N_WORKERS = N_CORES * N_SUBCORES

N_SENT = 4096
SENT_LEN = 50
SENT_PAD = 56
SENT_PER_W = N_SENT // N_WORKERS
PAIRS_PER_W = SENT_PER_W // 2
NBUF = 4
SKEW = 2

_MESH = plsc.VectorSubcoreMesh(core_axis_name="c", subcore_axis_name="s")


@functools.partial(
    pl.kernel,
    out_type=jax.ShapeDtypeStruct((N_SENT, SENT_PAD, DIM), jnp.float32),
    mesh=_MESH,
    scratch_types=[
        pltpu.VMEM((SENT_PER_W, SENT_LEN), jnp.int32),
        pltpu.VMEM((NBUF, 2, SENT_PAD, DIM), jnp.float32),
        pltpu.SemaphoreType.DMA((NBUF,)),
        pltpu.SemaphoreType.DMA((NBUF,)),
    ],
)
def _gather_kernel(idx_hbm, table_hbm, out_hbm, idx_v, rows_v, gsem, ssem):
    wid = lax.axis_index("s") * N_CORES + lax.axis_index("c")
    sent_base = wid * SENT_PER_W
    pltpu.sync_copy(idx_hbm.at[pl.ds(sent_base, SENT_PER_W)], idx_v)

    def gathers(p):
        b = lax.rem(p, NBUF) if not isinstance(p, int) else p % NBUF
        return [
            pltpu.make_async_copy(
                table_hbm.at[idx_v.at[2 * p + k]],
                rows_v.at[b, k, pl.ds(0, SENT_LEN)],
                gsem.at[b])
            for k in range(2)
        ]

    def store(p):
        b = lax.rem(p, NBUF) if not isinstance(p, int) else p % NBUF
        return pltpu.make_async_copy(
            rows_v.at[b],
            out_hbm.at[pl.ds(sent_base + 2 * p, 2)],
            ssem.at[b])

    def start_gathers(p):
        for g in gathers(p):
            g.start()

    def wait_gathers(p):
        for g in gathers(p):
            g.wait()

    for i in range(SKEW):
        start_gathers(i)
    for i in range(SKEW, NBUF):
        wait_gathers(i - SKEW)
        store(i - SKEW).start()
        start_gathers(i)

    def steady(i, _):
        store(i - NBUF).wait()
        wait_gathers(i - SKEW)
        store(i - SKEW).start()
        start_gathers(i)
        return 0

    lax.fori_loop(NBUF, PAIRS_PER_W, steady, 0, unroll=False)

    for i in range(PAIRS_PER_W, PAIRS_PER_W + SKEW):
        store(i - NBUF).wait()
        wait_gathers(i - SKEW)
        store(i - SKEW).start()
    for p in range(PAIRS_PER_W - NBUF + SKEW, PAIRS_PER_W):
        store(p).wait()


def kernel(indices, table):
    idx = indices.astype(jnp.int32)
    out = _gather_kernel(idx, table)
    return out[:, :SENT_LEN, :]


# speedup vs baseline: 5.9594x; 1.1709x over previous
import functools

import jax
import jax.numpy as jnp
from jax import lax
from jax.experimental import pallas as pl
from jax.experimental.pallas import tpu as pltpu, tpu_sc as plsc

VOCAB = 100000
DIM = 128

N_CORES = 2
N_SUBCORES = 16
N_WORKERS = N_CORES * N_SUBCORES

N_SENT = 4096
SENT_LEN = 50
SENT_PAD = 56
SENT_PER_W = N_SENT // N_WORKERS
PAIRS_PER_W = SENT_PER_W // 2
NBUF = 4
SKEW = 2

_MESH = plsc.VectorSubcoreMesh(core_axis_name="c", subcore_axis_name="s")


@functools.partial(
    pl.kernel,
    out_type=jax.ShapeDtypeStruct((N_SENT, SENT_LEN, DIM), jnp.float32),
    mesh=_MESH,
    compiler_params=pltpu.CompilerParams(use_tc_tiling_on_sc=True),
    scratch_types=[
        pltpu.VMEM((SENT_PER_W, SENT_LEN), jnp.int32),
        pltpu.VMEM((NBUF, 2, SENT_LEN, DIM), jnp.float32),
        pltpu.SemaphoreType.DMA((NBUF,)),
        pltpu.SemaphoreType.DMA((NBUF,)),
    ],
)
def _gather_kernel(idx_hbm, table_hbm, out_hbm, idx_v, rows_v, gsem, ssem):
    wid = lax.axis_index("s") * N_CORES + lax.axis_index("c")
    sent_base = wid * SENT_PER_W
    pltpu.sync_copy(idx_hbm.at[pl.ds(sent_base, SENT_PER_W)], idx_v)

    def gathers(p):
        b = lax.rem(p, NBUF) if not isinstance(p, int) else p % NBUF
        return [
            pltpu.make_async_copy(
                table_hbm.at[idx_v.at[2 * p + k]],
                rows_v.at[b, k],
                gsem.at[b])
            for k in range(2)
        ]

    def store(p):
        b = lax.rem(p, NBUF) if not isinstance(p, int) else p % NBUF
        return pltpu.make_async_copy(
            rows_v.at[b],
            out_hbm.at[pl.ds(sent_base + 2 * p, 2)],
            ssem.at[b])

    def start_gathers(p):
        for g in gathers(p):
            g.start()

    def wait_gathers(p):
        for g in gathers(p):
            g.wait()

    for i in range(SKEW):
        start_gathers(i)
    for i in range(SKEW, NBUF):
        wait_gathers(i - SKEW)
        store(i - SKEW).start()
        start_gathers(i)

    def steady(i, _):
        store(i - NBUF).wait()
        wait_gathers(i - SKEW)
        store(i - SKEW).start()
        start_gathers(i)
        return 0

    lax.fori_loop(NBUF, PAIRS_PER_W, steady, 0, unroll=False)

    for i in range(PAIRS_PER_W, PAIRS_PER_W + SKEW):
        store(i - NBUF).wait()
        wait_gathers(i - SKEW)
        store(i - SKEW).start()
    for p in range(PAIRS_PER_W - NBUF + SKEW, PAIRS_PER_W):
        store(p).wait()


def kernel(indices, table):
    idx = indices.astype(jnp.int32)
    return _gather_kernel(idx, table)


# speedup vs baseline: 10.6961x; 1.7948x over previous
import functools

import jax
import jax.numpy as jnp
from jax import lax
from jax.experimental import pallas as pl
from jax.experimental.pallas import tpu as pltpu, tpu_sc as plsc

VOCAB = 100000
DIM = 128

N_CORES = 2
N_SUBCORES = 16
N_WORKERS = N_CORES * N_SUBCORES

N_SENT = 4096
SENT_LEN = 50
SENT_PER_W = N_SENT // N_WORKERS
NBUF = 6
SKEW = 3

_MESH = plsc.VectorSubcoreMesh(core_axis_name="c", subcore_axis_name="s")


@functools.partial(
    pl.kernel,
    out_type=jax.ShapeDtypeStruct((SENT_LEN, N_SENT, DIM), jnp.float32),
    mesh=_MESH,
    scratch_types=[
        pltpu.VMEM((SENT_LEN, SENT_PER_W), jnp.int32),
        pltpu.VMEM((NBUF, SENT_PER_W, DIM), jnp.float32),
        pltpu.SemaphoreType.DMA((NBUF,)),
        pltpu.SemaphoreType.DMA((NBUF,)),
    ],
)
def _gather_kernel(idx_hbm, table_hbm, out_hbm, idx_v, rows_v, gsem, ssem):
    wid = lax.axis_index("s") * N_CORES + lax.axis_index("c")
    sent_base = wid * SENT_PER_W
    pltpu.sync_copy(idx_hbm.at[wid], idx_v)

    def gather(j):
        b = lax.rem(j, NBUF) if not isinstance(j, int) else j % NBUF
        return pltpu.make_async_copy(
            table_hbm.at[idx_v.at[j]], rows_v.at[b], gsem.at[b])

    def store(j):
        b = lax.rem(j, NBUF) if not isinstance(j, int) else j % NBUF
        return pltpu.make_async_copy(
            rows_v.at[b], out_hbm.at[j, pl.ds(sent_base, SENT_PER_W)],
            ssem.at[b])

    for i in range(SKEW):
        gather(i).start()
    for i in range(SKEW, NBUF):
        gather(i - SKEW).wait()
        store(i - SKEW).start()
        gather(i).start()

    def steady(i, _):
        store(i - NBUF).wait()
        gather(i - SKEW).wait()
        store(i - SKEW).start()
        gather(i).start()
        return 0

    lax.fori_loop(NBUF, SENT_LEN, steady, 0, unroll=False)

    for i in range(SENT_LEN, SENT_LEN + SKEW):
        store(i - NBUF).wait()
        gather(i - SKEW).wait()
        store(i - SKEW).start()
    for j in range(SENT_LEN - NBUF + SKEW, SENT_LEN):
        store(j).wait()


def kernel(indices, table):
    idx = (indices.astype(jnp.int32)
           .reshape(N_WORKERS, SENT_PER_W, SENT_LEN)
           .transpose(0, 2, 1))
    out = _gather_kernel(idx, table)
    return out.transpose(1, 0, 2)


# speedup vs baseline: 10.7622x; 1.0062x over previous
import functools

import jax
import jax.numpy as jnp
from jax import lax
from jax.experimental import pallas as pl
from jax.experimental.pallas import tpu as pltpu, tpu_sc as plsc

VOCAB = 100000
DIM = 128

N_CORES = 2
N_SUBCORES = 16
N_WORKERS = N_CORES * N_SUBCORES

N_SENT = 4096
SENT_LEN = 50
SENT_PER_W = N_SENT // N_WORKERS
NBUF = 6
SKEW = 3

_MESH = plsc.VectorSubcoreMesh(core_axis_name="c", subcore_axis_name="s")


@functools.partial(
    pl.kernel,
    out_type=jax.ShapeDtypeStruct((SENT_LEN, N_SENT, DIM), jnp.float32),
    mesh=_MESH,
    scratch_types=[
        pltpu.VMEM((SENT_LEN, SENT_PER_W), jnp.int32),
        pltpu.VMEM((NBUF, SENT_PER_W, DIM), jnp.float32),
        pltpu.SemaphoreType.DMA((NBUF,)),
        pltpu.SemaphoreType.DMA((NBUF,)),
    ],
)
def _gather_kernel(idx_hbm, table_hbm, out_hbm, idx_v, rows_v, gsem, ssem):
    wid = lax.axis_index("s") * N_CORES + lax.axis_index("c")
    sent_base = wid * SENT_PER_W
    pltpu.sync_copy(
        idx_hbm.at[pl.ds(0, SENT_LEN), pl.ds(sent_base, SENT_PER_W)], idx_v)

    def gather(j):
        b = lax.rem(j, NBUF) if not isinstance(j, int) else j % NBUF
        return pltpu.make_async_copy(
            table_hbm.at[idx_v.at[j]], rows_v.at[b], gsem.at[b])

    def store(j):
        b = lax.rem(j, NBUF) if not isinstance(j, int) else j % NBUF
        return pltpu.make_async_copy(
            rows_v.at[b], out_hbm.at[j, pl.ds(sent_base, SENT_PER_W)],
            ssem.at[b])

    for i in range(SKEW):
        gather(i).start()
    for i in range(SKEW, NBUF):
        gather(i - SKEW).wait()
        store(i - SKEW).start()
        gather(i).start()

    def steady(i, _):
        store(i - NBUF).wait()
        gather(i - SKEW).wait()
        store(i - SKEW).start()
        gather(i).start()
        return 0

    lax.fori_loop(NBUF, SENT_LEN, steady, 0, unroll=False)

    for i in range(SENT_LEN, SENT_LEN + SKEW):
        store(i - NBUF).wait()
        gather(i - SKEW).wait()
        store(i - SKEW).start()
    for j in range(SENT_LEN - NBUF + SKEW, SENT_LEN):
        store(j).wait()


def kernel(indices, table):
    idx = indices.astype(jnp.int32).T
    out = _gather_kernel(idx, table)
    return out.transpose(1, 0, 2)
